# baseline (speedup 1.0000x reference)
import functools

import jax
import jax.numpy as jnp
from jax import lax
from jax.experimental import pallas as pl
from jax.experimental.pallas import tpu as pltpu
from jax.experimental.pallas import tpu_sc as plsc

BATCH = 16384
EMBED_DIM = 64
GATHER_WINDOW = 128
MLP_BLOCK = 2048


def _sc_gather4(tables, idxs):
    mesh = plsc.VectorSubcoreMesh(core_axis_name="core", subcore_axis_name="subcore")
    out_type = [jax.ShapeDtypeStruct((BATCH, EMBED_DIM), jnp.float32)] * 4

    @functools.partial(
        pl.kernel, out_type=out_type, mesh=mesh,
        compiler_params=pltpu.CompilerParams(use_tc_tiling_on_sc=False))
    def gather_kernel(t0, t1, t2, t3, i0, i1, i2, i3, o0, o1, o2, o3):
        for table_hbm, idx_hbm, out_hbm in ((t0, i0, o0), (t1, i1, o1),
                                            (t2, i2, o2), (t3, i3, o3)):
            def body(i_vmem, o_vmem, table=table_hbm):
                pltpu.sync_copy(table.at[i_vmem.at[0]], o_vmem)

            pltpu.emit_pipeline(
                body,
                grid=(BATCH // GATHER_WINDOW,),
                in_specs=[pl.BlockSpec((1, GATHER_WINDOW),
                                       index_map=lambda i: (0, i))],
                out_specs=[pl.BlockSpec((GATHER_WINDOW, EMBED_DIM),
                                        index_map=lambda i: (i, 0))],
                core_axis_name=("core", "subcore"),
                dimension_semantics=(pltpu.PARALLEL,),
            )(idx_hbm, out_hbm)

    return gather_kernel(*tables, *idxs)


def _mlp_body(u_ref, r_ref, i_ref, n_ref, w1_ref, b1_ref, w2_ref, b2_ref, o_ref):
    x = jnp.concatenate([u_ref[...], r_ref[...], i_ref[...], n_ref[...]], axis=1)
    h = jnp.dot(x, w1_ref[...], preferred_element_type=jnp.float32) + b1_ref[...]
    h = jnp.maximum(h, 0.0)
    o_ref[...] = jnp.dot(h, w2_ref[...], preferred_element_type=jnp.float32) + b2_ref[...]


def _tc_mlp(u, r, i, n, W1, b1, W2, b2):
    d4, d2, d1 = 4 * EMBED_DIM, 2 * EMBED_DIM, EMBED_DIM
    emb_spec = pl.BlockSpec((MLP_BLOCK, d1), lambda g: (g, 0))
    return pl.pallas_call(
        _mlp_body,
        grid=(BATCH // MLP_BLOCK,),
        in_specs=[
            emb_spec, emb_spec, emb_spec, emb_spec,
            pl.BlockSpec((d4, d2), lambda g: (0, 0)),
            pl.BlockSpec((1, d2), lambda g: (0, 0)),
            pl.BlockSpec((d2, d1), lambda g: (0, 0)),
            pl.BlockSpec((1, d1), lambda g: (0, 0)),
        ],
        out_specs=pl.BlockSpec((MLP_BLOCK, d1), lambda g: (g, 0)),
        out_shape=jax.ShapeDtypeStruct((BATCH, d1), jnp.float32),
    )(u, r, i, n, W1, b1.reshape(1, d2), W2, b2.reshape(1, d1))


@jax.jit
def kernel(user_idx, recipe_idx, ingredient_idx, nutrition_idx,
           user_table, recipe_table, ingredient_table, nutrition_table,
           W1, b1, W2, b2):
    idxs = [x.astype(jnp.int32).reshape(1, BATCH)
            for x in (user_idx, recipe_idx, ingredient_idx, nutrition_idx)]
    tables = (user_table, recipe_table, ingredient_table, nutrition_table)
    u, r, i, n = _sc_gather4(tables, idxs)
    return _tc_mlp(u, r, i, n, W1, b1, W2, b2)
